# baseline (speedup 1.0000x reference)
import functools

import jax
import jax.numpy as jnp
from jax.experimental import pallas as pl
from jax.experimental.pallas import tpu as pltpu
from jax.experimental.pallas import tpu_sc as plsc

N_TOK = 4608
D = 256
K = 8192
BETA = 0.01
KT = 1024
NB = 576
NEG_INF = -3.0e38


def _xla_rownorm(x):
    n = jnp.linalg.norm(x, axis=1, keepdims=True)
    return x / jnp.maximum(n, 1e-12)


NK = K // KT


def _argmax_consume(d, base, vmax_ref, vidx_ref):
    lane = jax.lax.broadcasted_iota(jnp.int32, (N_TOK, 128), 1)
    for g in range(KT // 128):
        dg = d[:, g * 128:(g + 1) * 128]
        upd = dg >= vmax_ref[...]
        vidx_ref[...] = jnp.where(upd, lane + (base + g * 128), vidx_ref[...])
        vmax_ref[...] = jnp.maximum(dg, vmax_ref[...])


def _argmax_body(zb_ref, wb_ref, idx_ref, d0_ref, d1_ref, vmax_ref, vidx_ref):
    k = pl.program_id(0)

    @pl.when(k == 0)
    def _init():
        vmax_ref[...] = jnp.full((N_TOK, 128), NEG_INF, jnp.float32)
        vidx_ref[...] = jnp.zeros((N_TOK, 128), jnp.int32)

    def _dot(out_ref):
        out_ref[...] = jax.lax.dot_general(
            zb_ref[...], wb_ref[...],
            (((1,), (1,)), ((), ())),
            preferred_element_type=jnp.float32,
        )

    @pl.when(jnp.logical_and(k < NK, k % 2 == 0))
    def _dot0():
        _dot(d0_ref)

    @pl.when(jnp.logical_and(k < NK, k % 2 == 1))
    def _dot1():
        _dot(d1_ref)

    @pl.when(jnp.logical_and(k > 0, k % 2 == 1))
    def _consume0():
        _argmax_consume(d0_ref[...], (k - 1) * KT, vmax_ref, vidx_ref)

    @pl.when(jnp.logical_and(k > 0, k % 2 == 0))
    def _consume1():
        _argmax_consume(d1_ref[...], (k - 1) * KT, vmax_ref, vidx_ref)

    @pl.when(k == NK)
    def _fin():
        vm = vmax_ref[...]
        m1 = jnp.max(vm, axis=1, keepdims=True)
        cand = jnp.where(vm == m1, vidx_ref[...], -1)
        idx_ref[...] = jnp.max(cand, axis=1, keepdims=True)


def _run_argmax(zb, wb, interpret=False):
    return pl.pallas_call(
        _argmax_body,
        grid=(NK + 1,),
        in_specs=[
            pl.BlockSpec((N_TOK, D), lambda k: (0, 0)),
            pl.BlockSpec((KT, D), lambda k: (jnp.minimum(k, NK - 1), 0)),
        ],
        out_specs=[
            pl.BlockSpec((N_TOK, 1), lambda k: (0, 0)),
        ],
        out_shape=[
            jax.ShapeDtypeStruct((N_TOK, 1), jnp.int32),
        ],
        scratch_shapes=[
            pltpu.VMEM((N_TOK, KT), jnp.float32),
            pltpu.VMEM((N_TOK, KT), jnp.float32),
            pltpu.VMEM((N_TOK, 128), jnp.float32),
            pltpu.VMEM((N_TOK, 128), jnp.int32),
        ],
        interpret=interpret,
    )(zb, wb)


GATHER_WIN = 128


def _sc_gather(w, idx_flat):
    mesh = plsc.VectorSubcoreMesh(core_axis_name="core", subcore_axis_name="subcore")

    @pl.kernel(out_type=jax.ShapeDtypeStruct((N_TOK, D), jnp.float32), mesh=mesh)
    def gather_kernel(w_hbm, i_hbm, o_hbm):
        def body(i_vmem, o_vmem):
            pltpu.sync_copy(w_hbm.at[i_vmem.at[0]], o_vmem)

        pltpu.emit_pipeline(
            body,
            grid=(N_TOK // GATHER_WIN,),
            in_specs=[pl.BlockSpec((1, GATHER_WIN), index_map=lambda i: (0, i))],
            out_specs=[pl.BlockSpec((GATHER_WIN, D), index_map=lambda i: (i, 0))],
            core_axis_name="subcore",
            dimension_semantics=(pltpu.PARALLEL,),
        )(i_hbm, o_hbm)

    return gather_kernel(w, idx_flat)


def _onehot_body(idx_ref, enc_ref, cnt_ref, acc_ref):
    i = pl.program_id(0)
    col = jax.lax.broadcasted_iota(jnp.int32, (NB, K), 1)
    oh = jnp.where(idx_ref[...] == col, 1.0, 0.0).astype(jnp.float32)
    enc_ref[...] = oh
    part = jnp.sum(oh, axis=0, keepdims=True)

    @pl.when(i == 0)
    def _init():
        acc_ref[...] = part

    @pl.when(i > 0)
    def _acc():
        acc_ref[...] += part

    @pl.when(i == pl.num_programs(0) - 1)
    def _fin():
        cnt_ref[...] = acc_ref[...]


def _run_onehot(idx, interpret=False):
    return pl.pallas_call(
        _onehot_body,
        grid=(N_TOK // NB,),
        in_specs=[pl.BlockSpec((NB, 1), lambda i: (i, 0))],
        out_specs=[
            pl.BlockSpec((NB, K), lambda i: (i, 0)),
            pl.BlockSpec((1, K), lambda i: (0, 0)),
        ],
        out_shape=[
            jax.ShapeDtypeStruct((N_TOK, K), jnp.float32),
            jax.ShapeDtypeStruct((1, K), jnp.float32),
        ],
        scratch_shapes=[pltpu.VMEM((1, K), jnp.float32)],
        interpret=interpret,
    )(idx)


def _final_body(zq_raw_ref, zn_ref, cnt_ref, zq_ref, loss_ref, ppx_ref):
    zn = zn_ref[...]
    x = zq_raw_ref[...]
    s = jnp.sum(x * x, axis=1, keepdims=True)
    zqn = x * jax.lax.rsqrt(jnp.maximum(s, 1e-24))
    zq_ref[...] = zn + (zqn - zn)
    diff = zqn - zn
    m = jnp.sum(diff * diff) / (N_TOK * D)
    loss_ref[0, 0] = BETA * m + m
    p = cnt_ref[...] / N_TOK
    ppx_ref[0, 0] = jnp.exp(-jnp.sum(p * jnp.log(p + 1e-10)))


def _run_final(zq_raw, zn, cnt, interpret=False):
    return pl.pallas_call(
        _final_body,
        in_specs=[
            pl.BlockSpec((N_TOK, D), lambda: (0, 0)),
            pl.BlockSpec((N_TOK, D), lambda: (0, 0)),
            pl.BlockSpec((1, K), lambda: (0, 0)),
        ],
        out_specs=[
            pl.BlockSpec((N_TOK, D), lambda: (0, 0)),
            pl.BlockSpec((1, 1), memory_space=pltpu.SMEM),
            pl.BlockSpec((1, 1), memory_space=pltpu.SMEM),
        ],
        out_shape=[
            jax.ShapeDtypeStruct((N_TOK, D), jnp.float32),
            jax.ShapeDtypeStruct((1, 1), jnp.float32),
            jax.ShapeDtypeStruct((1, 1), jnp.float32),
        ],
        interpret=interpret,
    )(zq_raw, zn, cnt)


def kernel(z, W):
    zt = jnp.transpose(z, (0, 2, 3, 1)).reshape(N_TOK, D)
    zn = _xla_rownorm(zt)
    znn = _xla_rownorm(zn)
    wnn = _xla_rownorm(_xla_rownorm(W))
    (idx,) = _run_argmax(znn.astype(jnp.bfloat16), wnn.astype(jnp.bfloat16))
    zq_raw = _sc_gather(W, idx.reshape(1, N_TOK))
    encodings, counts = _run_onehot(idx)
    zq, loss, ppx = _run_final(zq_raw, zn, counts)
    z_q = jnp.transpose(zq.reshape(8, 24, 24, D), (0, 3, 1, 2))
    return (z_q, idx.reshape(N_TOK), loss.reshape(()), encodings, ppx.reshape(()))


# speedup vs baseline: 1.4398x; 1.4398x over previous
import functools

import jax
import jax.numpy as jnp
from jax.experimental import pallas as pl
from jax.experimental.pallas import tpu as pltpu
from jax.experimental.pallas import tpu_sc as plsc

N_TOK = 4608
D = 256
K = 8192
BETA = 0.01
KT = 1024
NB = 576
NEG_INF = -3.0e38


def _xla_rownorm(x):
    n = jnp.linalg.norm(x, axis=1, keepdims=True)
    return x / jnp.maximum(n, 1e-12)


def _argmax_body(zb_ref, wb_ref, idx_ref, idxr_ref, vmax_ref, vidx_ref):
    k = pl.program_id(0)

    @pl.when(k == 0)
    def _init():
        vmax_ref[...] = jnp.full((N_TOK, 128), NEG_INF, jnp.float32)
        vidx_ref[...] = jnp.zeros((N_TOK, 128), jnp.int32)

    d = jax.lax.dot_general(
        zb_ref[...], wb_ref[...],
        (((1,), (1,)), ((), ())),
        preferred_element_type=jnp.float32,
    )
    lane = jax.lax.broadcasted_iota(jnp.int32, (N_TOK, 128), 1)
    for g in range(KT // 128):
        dg = d[:, g * 128:(g + 1) * 128]
        upd = dg >= vmax_ref[...]
        vidx_ref[...] = jnp.where(upd, lane + (k * KT + g * 128), vidx_ref[...])
        vmax_ref[...] = jnp.maximum(dg, vmax_ref[...])

    @pl.when(k == pl.num_programs(0) - 1)
    def _fin():
        vm = vmax_ref[...]
        m1 = jnp.max(vm, axis=1, keepdims=True)
        cand = jnp.where(vm == m1, vidx_ref[...], -1)
        idxc = jnp.max(cand, axis=1, keepdims=True)
        idx_ref[...] = idxc
        idxr_ref[...] = idxc.reshape(1, N_TOK)


def _run_argmax(zb, wb, interpret=False):
    return pl.pallas_call(
        _argmax_body,
        grid=(K // KT,),
        in_specs=[
            pl.BlockSpec((N_TOK, D), lambda k: (0, 0)),
            pl.BlockSpec((KT, D), lambda k: (k, 0)),
        ],
        out_specs=[
            pl.BlockSpec((N_TOK, 1), lambda k: (0, 0)),
            pl.BlockSpec((1, N_TOK), lambda k: (0, 0)),
        ],
        out_shape=[
            jax.ShapeDtypeStruct((N_TOK, 1), jnp.int32),
            jax.ShapeDtypeStruct((1, N_TOK), jnp.int32),
        ],
        scratch_shapes=[
            pltpu.VMEM((N_TOK, 128), jnp.float32),
            pltpu.VMEM((N_TOK, 128), jnp.int32),
        ],
        interpret=interpret,
    )(zb, wb)


GATHER_WIN = 128


def _sc_gather(w, idx_flat):
    mesh = plsc.VectorSubcoreMesh(core_axis_name="core", subcore_axis_name="subcore")

    @pl.kernel(out_type=jax.ShapeDtypeStruct((N_TOK, D), jnp.float32), mesh=mesh)
    def gather_kernel(w_hbm, i_hbm, o_hbm):
        def body(i_vmem, o_vmem):
            pltpu.sync_copy(w_hbm.at[i_vmem.at[0]], o_vmem)

        pltpu.emit_pipeline(
            body,
            grid=(N_TOK // GATHER_WIN,),
            in_specs=[pl.BlockSpec((1, GATHER_WIN), index_map=lambda i: (0, i))],
            out_specs=[pl.BlockSpec((GATHER_WIN, D), index_map=lambda i: (i, 0))],
            core_axis_name="subcore",
            dimension_semantics=(pltpu.PARALLEL,),
        )(i_hbm, o_hbm)

    return gather_kernel(w, idx_flat)


def _onehot_body(idx_ref, enc_ref, ppx_ref, acc_ref):
    i = pl.program_id(0)
    col = jax.lax.broadcasted_iota(jnp.int32, (NB, K), 1)
    oh = jnp.where(idx_ref[...] == col, 1.0, 0.0).astype(jnp.float32)
    enc_ref[...] = oh
    part = jnp.sum(oh, axis=0, keepdims=True)

    @pl.when(i == 0)
    def _init():
        acc_ref[...] = part

    @pl.when(i > 0)
    def _acc():
        acc_ref[...] += part

    @pl.when(i == pl.num_programs(0) - 1)
    def _fin():
        p = acc_ref[...] / N_TOK
        ppx_ref[0, 0] = jnp.exp(-jnp.sum(p * jnp.log(p + 1e-10)))


def _run_onehot(idx, interpret=False):
    return pl.pallas_call(
        _onehot_body,
        grid=(N_TOK // NB,),
        in_specs=[pl.BlockSpec((NB, 1), lambda i: (i, 0))],
        out_specs=[
            pl.BlockSpec((NB, K), lambda i: (i, 0)),
            pl.BlockSpec((1, 1), lambda i: (0, 0), memory_space=pltpu.SMEM),
        ],
        out_shape=[
            jax.ShapeDtypeStruct((N_TOK, K), jnp.float32),
            jax.ShapeDtypeStruct((1, 1), jnp.float32),
        ],
        scratch_shapes=[pltpu.VMEM((1, K), jnp.float32)],
        interpret=interpret,
    )(idx)


def _final_body(zq_raw_ref, zn_ref, zq_ref, loss_ref):
    zn = zn_ref[...]
    x = zq_raw_ref[...]
    s = jnp.sum(x * x, axis=1, keepdims=True)
    zqn = x * jax.lax.rsqrt(jnp.maximum(s, 1e-24))
    zq_ref[...] = zn + (zqn - zn)
    diff = zqn - zn
    m = jnp.sum(diff * diff) / (N_TOK * D)
    loss_ref[0, 0] = BETA * m + m


def _run_final(zq_raw, zn, interpret=False):
    return pl.pallas_call(
        _final_body,
        in_specs=[
            pl.BlockSpec((N_TOK, D), lambda: (0, 0)),
            pl.BlockSpec((N_TOK, D), lambda: (0, 0)),
        ],
        out_specs=[
            pl.BlockSpec((N_TOK, D), lambda: (0, 0)),
            pl.BlockSpec((1, 1), memory_space=pltpu.SMEM),
        ],
        out_shape=[
            jax.ShapeDtypeStruct((N_TOK, D), jnp.float32),
            jax.ShapeDtypeStruct((1, 1), jnp.float32),
        ],
        interpret=interpret,
    )(zq_raw, zn)


def kernel(z, W):
    zt = jnp.transpose(z, (0, 2, 3, 1)).reshape(N_TOK, D)
    zn = _xla_rownorm(zt)
    znn = _xla_rownorm(zn)
    wnn = _xla_rownorm(_xla_rownorm(W))
    idx, idxr = _run_argmax(znn.astype(jnp.bfloat16), wnn.astype(jnp.bfloat16))
    zq_raw = _sc_gather(W, idxr)
    encodings, ppx = _run_onehot(idx)
    zq, loss = _run_final(zq_raw, zn)
    z_q = jnp.transpose(zq.reshape(8, 24, 24, D), (0, 3, 1, 2))
    return (z_q, idx.reshape(N_TOK), loss.reshape(()), encodings, ppx.reshape(()))


# speedup vs baseline: 1.4618x; 1.0153x over previous
import functools

import jax
import jax.numpy as jnp
from jax.experimental import pallas as pl
from jax.experimental.pallas import tpu as pltpu
from jax.experimental.pallas import tpu_sc as plsc

N_TOK = 4608
D = 256
K = 8192
BETA = 0.01
KT = 1024
NB = 576
NEG_INF = -3.0e38


def _xla_rownorm(x):
    n = jnp.linalg.norm(x, axis=1, keepdims=True)
    return x / jnp.maximum(n, 1e-12)


def _argmax_body(zb_ref, wb_ref, idx_ref, idxr_ref, vmax_ref, vidx_ref):
    k = pl.program_id(0)

    @pl.when(k == 0)
    def _init():
        vmax_ref[...] = jnp.full((N_TOK, 128), NEG_INF, jnp.float32)
        vidx_ref[...] = jnp.zeros((N_TOK, 128), jnp.int32)

    d = jax.lax.dot_general(
        zb_ref[...], wb_ref[...],
        (((1,), (1,)), ((), ())),
        preferred_element_type=jnp.float32,
    )
    lane = jax.lax.broadcasted_iota(jnp.int32, (N_TOK, 128), 1)
    for g in range(KT // 128):
        dg = d[:, g * 128:(g + 1) * 128]
        upd = dg >= vmax_ref[...]
        vidx_ref[...] = jnp.where(upd, lane + (k * KT + g * 128), vidx_ref[...])
        vmax_ref[...] = jnp.maximum(dg, vmax_ref[...])

    @pl.when(k == pl.num_programs(0) - 1)
    def _fin():
        vm = vmax_ref[...]
        m1 = jnp.max(vm, axis=1, keepdims=True)
        cand = jnp.where(vm == m1, vidx_ref[...], -1)
        idxc = jnp.max(cand, axis=1, keepdims=True)
        idx_ref[...] = idxc
        idxr_ref[...] = idxc.reshape(1, N_TOK)


def _run_argmax(zb, wb, interpret=False):
    return pl.pallas_call(
        _argmax_body,
        grid=(K // KT,),
        in_specs=[
            pl.BlockSpec((N_TOK, D), lambda k: (0, 0)),
            pl.BlockSpec((KT, D), lambda k: (k, 0)),
        ],
        out_specs=[
            pl.BlockSpec((N_TOK, 1), lambda k: (0, 0)),
            pl.BlockSpec((1, N_TOK), lambda k: (0, 0)),
        ],
        out_shape=[
            jax.ShapeDtypeStruct((N_TOK, 1), jnp.int32),
            jax.ShapeDtypeStruct((1, N_TOK), jnp.int32),
        ],
        scratch_shapes=[
            pltpu.VMEM((N_TOK, 128), jnp.float32),
            pltpu.VMEM((N_TOK, 128), jnp.int32),
        ],
        interpret=interpret,
    )(zb, wb)


GATHER_BPW = N_TOK // 32
GATHER_HALF = GATHER_BPW // 2


def _sc_gather(w, idx_flat):
    mesh = plsc.VectorSubcoreMesh(core_axis_name="c", subcore_axis_name="s")

    @functools.partial(
        pl.kernel, mesh=mesh,
        out_type=jax.ShapeDtypeStruct((N_TOK, D), jnp.float32),
        scratch_types=[
            pltpu.VMEM((GATHER_BPW,), jnp.int32),
            pltpu.VMEM((GATHER_BPW, D), jnp.float32),
            pltpu.SemaphoreType.DMA,
        ],
    )
    def gather_kernel(w_hbm, i_hbm, o_hbm, idx_v, rows_v, sem):
        wid = jax.lax.axis_index("s") * 2 + jax.lax.axis_index("c")
        base = wid * GATHER_BPW
        pltpu.sync_copy(i_hbm.at[pl.ds(base, GATHER_BPW)], idx_v)
        pltpu.async_copy(w_hbm.at[idx_v.at[pl.ds(0, GATHER_HALF)]],
                         rows_v.at[pl.ds(0, GATHER_HALF)], sem).wait()
        pltpu.async_copy(w_hbm.at[idx_v.at[pl.ds(GATHER_HALF, GATHER_HALF)]],
                         rows_v.at[pl.ds(GATHER_HALF, GATHER_HALF)], sem).wait()
        pltpu.sync_copy(rows_v, o_hbm.at[pl.ds(base, GATHER_BPW)])

    return gather_kernel(w, idx_flat)


def _onehot_body(idx_ref, enc_ref, ppx_ref, acc_ref):
    i = pl.program_id(0)
    col = jax.lax.broadcasted_iota(jnp.int32, (NB, K), 1)
    oh = jnp.where(idx_ref[...] == col, 1.0, 0.0).astype(jnp.float32)
    enc_ref[...] = oh
    part = jnp.sum(oh, axis=0, keepdims=True)

    @pl.when(i == 0)
    def _init():
        acc_ref[...] = part

    @pl.when(i > 0)
    def _acc():
        acc_ref[...] += part

    @pl.when(i == pl.num_programs(0) - 1)
    def _fin():
        p = acc_ref[...] / N_TOK
        ppx_ref[0, 0] = jnp.exp(-jnp.sum(p * jnp.log(p + 1e-10)))


def _run_onehot(idx, interpret=False):
    return pl.pallas_call(
        _onehot_body,
        grid=(N_TOK // NB,),
        in_specs=[pl.BlockSpec((NB, 1), lambda i: (i, 0))],
        out_specs=[
            pl.BlockSpec((NB, K), lambda i: (i, 0)),
            pl.BlockSpec((1, 1), lambda i: (0, 0), memory_space=pltpu.SMEM),
        ],
        out_shape=[
            jax.ShapeDtypeStruct((N_TOK, K), jnp.float32),
            jax.ShapeDtypeStruct((1, 1), jnp.float32),
        ],
        scratch_shapes=[pltpu.VMEM((1, K), jnp.float32)],
        interpret=interpret,
    )(idx)


def _final_body(zq_raw_ref, zn_ref, zq_ref, loss_ref):
    zn = zn_ref[...]
    x = zq_raw_ref[...]
    s = jnp.sum(x * x, axis=1, keepdims=True)
    zqn = x * jax.lax.rsqrt(jnp.maximum(s, 1e-24))
    zq_ref[...] = zn + (zqn - zn)
    diff = zqn - zn
    m = jnp.sum(diff * diff) / (N_TOK * D)
    loss_ref[0, 0] = BETA * m + m


def _run_final(zq_raw, zn, interpret=False):
    return pl.pallas_call(
        _final_body,
        in_specs=[
            pl.BlockSpec((N_TOK, D), lambda: (0, 0)),
            pl.BlockSpec((N_TOK, D), lambda: (0, 0)),
        ],
        out_specs=[
            pl.BlockSpec((N_TOK, D), lambda: (0, 0)),
            pl.BlockSpec((1, 1), memory_space=pltpu.SMEM),
        ],
        out_shape=[
            jax.ShapeDtypeStruct((N_TOK, D), jnp.float32),
            jax.ShapeDtypeStruct((1, 1), jnp.float32),
        ],
        interpret=interpret,
    )(zq_raw, zn)


def kernel(z, W):
    zt = jnp.transpose(z, (0, 2, 3, 1)).reshape(N_TOK, D)
    zn = _xla_rownorm(zt)
    znn = _xla_rownorm(zn)
    wnn = _xla_rownorm(_xla_rownorm(W))
    idx, idxr = _run_argmax(znn.astype(jnp.bfloat16), wnn.astype(jnp.bfloat16))
    zq_raw = _sc_gather(W, idxr.reshape(N_TOK))
    encodings, ppx = _run_onehot(idx)
    zq, loss = _run_final(zq_raw, zn)
    z_q = jnp.transpose(zq.reshape(8, 24, 24, D), (0, 3, 1, 2))
    return (z_q, idx.reshape(N_TOK), loss.reshape(()), encodings, ppx.reshape(()))
